# Initial kernel scaffold; baseline (speedup 1.0000x reference)
import jax
import jax.numpy as jnp
from jax.experimental import pallas as pl


def kernel(x, y, w1, b1, w2, b2):
    raise NotImplementedError("write your pallas kernel here")


# speedup vs baseline: 1.2573x; 1.2573x over previous
import functools

import jax
import jax.numpy as jnp
from jax.experimental import pallas as pl
from jax.experimental.pallas import tpu as pltpu


def _ca_fused_kernel(y_ref, x_ref, w1_ref, b1_ref, w2_ref, b2_ref, o_ref,
                     *, inv_hw):
    gap = jnp.sum(y_ref[...], axis=-1, keepdims=True,
                  dtype=jnp.float32) * inv_hw
    h = jax.lax.dot(w1_ref[...], gap,
                    preferred_element_type=jnp.float32) + b1_ref[...]
    h = jnp.maximum(h, 0.0)
    a = jax.lax.dot(w2_ref[...], h,
                    preferred_element_type=jnp.float32) + b2_ref[...]
    a = jax.nn.sigmoid(a)
    o_ref[...] = x_ref[...] * a.astype(x_ref.dtype)


def kernel(x, y, w1, b1, w2, b2):
    B, C, H, W = x.shape
    Cr = w1.shape[0]
    HW = H * W

    x2 = x.reshape(B, C, HW)
    y2 = y.reshape(B, C, HW)
    w1f = w1.astype(jnp.float32)
    w2f = w2.astype(jnp.float32)
    b1c = b1.reshape(Cr, 1).astype(jnp.float32)
    b2c = b2.reshape(C, 1).astype(jnp.float32)

    slab = pl.BlockSpec((pl.Squeezed(), C, HW), lambda b: (b, 0, 0))
    itemsize = x.dtype.itemsize
    cost = pl.CostEstimate(
        flops=int(B * (2 * C * HW + 4 * C * Cr)),
        transcendentals=int(B * C),
        bytes_accessed=int(3 * B * C * HW * itemsize),
    )
    out = pl.pallas_call(
        functools.partial(_ca_fused_kernel, inv_hw=1.0 / HW),
        out_shape=jax.ShapeDtypeStruct((B, C, HW), x.dtype),
        grid=(B,),
        in_specs=[
            slab,
            slab,
            pl.BlockSpec((Cr, C), lambda b: (0, 0)),
            pl.BlockSpec((Cr, 1), lambda b: (0, 0)),
            pl.BlockSpec((C, Cr), lambda b: (0, 0)),
            pl.BlockSpec((C, 1), lambda b: (0, 0)),
        ],
        out_specs=slab,
        compiler_params=pltpu.CompilerParams(
            dimension_semantics=("parallel",),
            vmem_limit_bytes=int(min(6 * C * HW * itemsize + (2 << 20),
                                     64 << 20)),
        ),
        cost_estimate=cost,
    )(y2, x2, w1f, b1c, w2f, b2c)
    return out.reshape(B, C, H, W)


# speedup vs baseline: 1.2729x; 1.0125x over previous
import functools

import jax
import jax.numpy as jnp
from jax.experimental import pallas as pl
from jax.experimental.pallas import tpu as pltpu


def _ca_fused_kernel(y_ref, x_ref, w1_ref, b1_ref, w2_ref, b2_ref, o_ref,
                     *, inv_hw, nb):
    gap = jnp.sum(y_ref[...], axis=-1, keepdims=True,
                  dtype=jnp.float32) * inv_hw
    for i in range(nb):
        h = jax.lax.dot(w1_ref[...], gap[i],
                        preferred_element_type=jnp.float32) + b1_ref[...]
        h = jnp.maximum(h, 0.0)
        a = jax.lax.dot(w2_ref[...], h,
                        preferred_element_type=jnp.float32) + b2_ref[...]
        a = jax.nn.sigmoid(a)
        o_ref[i] = x_ref[i] * a.astype(x_ref.dtype)


def kernel(x, y, w1, b1, w2, b2):
    B, C, H, W = x.shape
    Cr = w1.shape[0]
    HW = H * W

    x2 = x.reshape(B, C, HW)
    y2 = y.reshape(B, C, HW)
    w1f = w1.astype(jnp.float32)
    w2f = w2.astype(jnp.float32)
    b1c = b1.reshape(Cr, 1).astype(jnp.float32)
    b2c = b2.reshape(C, 1).astype(jnp.float32)

    NB = 1
    for cand in (4, 2):
        if B % cand == 0 and cand * C * HW * x.dtype.itemsize * 6 <= (56 << 20):
            NB = cand
            break

    slab = pl.BlockSpec((NB, C, HW), lambda b: (b, 0, 0))
    itemsize = x.dtype.itemsize
    cost = pl.CostEstimate(
        flops=int(B * (2 * C * HW + 4 * C * Cr)),
        transcendentals=int(B * C),
        bytes_accessed=int(3 * B * C * HW * itemsize),
    )
    out = pl.pallas_call(
        functools.partial(_ca_fused_kernel, inv_hw=1.0 / HW, nb=NB),
        out_shape=jax.ShapeDtypeStruct((B, C, HW), x.dtype),
        grid=(B // NB,),
        in_specs=[
            slab,
            slab,
            pl.BlockSpec((Cr, C), lambda b: (0, 0)),
            pl.BlockSpec((Cr, 1), lambda b: (0, 0)),
            pl.BlockSpec((C, Cr), lambda b: (0, 0)),
            pl.BlockSpec((C, 1), lambda b: (0, 0)),
        ],
        out_specs=slab,
        compiler_params=pltpu.CompilerParams(
            dimension_semantics=("parallel",),
            vmem_limit_bytes=int(min(6 * NB * C * HW * itemsize + (2 << 20),
                                     60 << 20)),
        ),
        cost_estimate=cost,
    )(y2, x2, w1f, b1c, w2f, b2c)
    return out.reshape(B, C, H, W)
